# Initial kernel scaffold; baseline (speedup 1.0000x reference)
import jax
import jax.numpy as jnp
from jax.experimental import pallas as pl


def kernel(roi_features, union_features, rel_pair_idxs, boxes_per_cls, W_obj, W_rel, freq_bias):
    raise NotImplementedError("write your pallas kernel here")


# speedup vs baseline: 16.1104x; 16.1104x over previous
import functools

import jax
import jax.numpy as jnp
from jax import lax
from jax.experimental import pallas as pl
from jax.experimental.pallas import tpu as pltpu
from jax.experimental.pallas import tpu_sc as plsc

_NUM_OBJ_CLS = 151
_NUM_REL_CLS = 51
_N_OBJ = 2000
_N_REL = 6000
_IN_CH = 4096
_NMS_THRESH = 0.5

_C_PAD = 152
_L_PAD = 2048
_R_PAD = 6144
_D_PAD = 64
_NEG = -1e30


def _mm_kernel(a_ref, w_ref, o_ref):
    o_ref[...] = jnp.dot(a_ref[...], w_ref[...],
                         preferred_element_type=jnp.float32)


def _matmul(a, w, block_rows):
    n, k = a.shape
    _, m = w.shape
    grid = n // block_rows
    return pl.pallas_call(
        _mm_kernel,
        grid=(grid,),
        in_specs=[
            pl.BlockSpec((block_rows, k), lambda i: (i, 0)),
            pl.BlockSpec((k, m), lambda i: (0, 0)),
        ],
        out_specs=pl.BlockSpec((block_rows, m), lambda i: (i, 0)),
        out_shape=jax.ShapeDtypeStruct((n, m), jnp.float32),
    )(a, w)


def _nms_kernel(logits_t_ref, x1_ref, y1_ref, x2_ref, y2_ref,
                labels_ref, probs_ref):
    cls_iota = lax.broadcasted_iota(jnp.int32, (_C_PAD, _L_PAD), 0)
    lane_iota = lax.broadcasted_iota(jnp.int32, (1, _L_PAD), 1)

    x = logits_t_ref[...]
    m0 = jnp.max(x, axis=0, keepdims=True)
    e = jnp.exp(x - m0)
    s = jnp.sum(e, axis=0, keepdims=True)
    p = e / s
    p = jnp.where(cls_iota == 0, 0.0, p)
    p = jnp.where(cls_iota >= _NUM_OBJ_CLS, 0.0, p)
    probs_ref[...] = p

    active0 = (lane_iota < _N_OBJ).astype(jnp.float32)
    labels0 = jnp.zeros((1, _L_PAD), jnp.int32)

    def body(_, carry):
        active, labels = carry
        probs = probs_ref[...]
        row_max = jnp.max(probs, axis=0, keepdims=True)
        masked = jnp.where(active > 0.0, row_max, -1.0)
        m = jnp.max(masked)
        box = jnp.min(jnp.where(masked == m, lane_iota, _L_PAD))
        row_arg = jnp.min(
            jnp.where(probs == row_max, cls_iota, _C_PAD), axis=0,
            keepdims=True)
        cls = jnp.min(jnp.where(lane_iota == box, row_arg, _C_PAD))

        is_box = lane_iota == box
        x1 = x1_ref[pl.ds(cls, 1), :]
        y1 = y1_ref[pl.ds(cls, 1), :]
        x2 = x2_ref[pl.ds(cls, 1), :]
        y2 = y2_ref[pl.ds(cls, 1), :]
        sx1 = jnp.max(jnp.where(is_box, x1, _NEG))
        sy1 = jnp.max(jnp.where(is_box, y1, _NEG))
        sx2 = jnp.max(jnp.where(is_box, x2, _NEG))
        sy2 = jnp.max(jnp.where(is_box, y2, _NEG))

        ix = jnp.maximum(jnp.minimum(sx2, x2) - jnp.maximum(sx1, x1), 0.0)
        iy = jnp.maximum(jnp.minimum(sy2, y2) - jnp.maximum(sy1, y1), 0.0)
        inter = ix * iy
        a1 = jnp.maximum(sx2 - sx1, 0.0) * jnp.maximum(sy2 - sy1, 0.0)
        a2 = jnp.maximum(x2 - x1, 0.0) * jnp.maximum(y2 - y1, 0.0)
        iou = inter / (a1 + a2 - inter + 1e-9)
        overlap = iou > _NMS_THRESH

        col = probs_ref[pl.ds(cls, 1), :]
        probs_ref[pl.ds(cls, 1), :] = jnp.where(overlap, 0.0, col)

        labels = jnp.where(is_box, cls, labels)
        active = jnp.where(is_box, 0.0, active)
        return active, labels

    _, labels = lax.fori_loop(0, _N_OBJ, body, (active0, labels0))
    labels_ref[...] = jnp.broadcast_to(labels, (8, _L_PAD))


def _nms(obj_logits, boxes_per_cls):
    logits_t = jnp.pad(obj_logits.T,
                       ((0, _C_PAD - _NUM_OBJ_CLS), (0, _L_PAD - _N_OBJ)),
                       constant_values=_NEG)
    bt = jnp.pad(boxes_per_cls.transpose(2, 1, 0),
                 ((0, 0), (0, _C_PAD - _NUM_OBJ_CLS), (0, _L_PAD - _N_OBJ)))
    labels8 = pl.pallas_call(
        _nms_kernel,
        out_shape=jax.ShapeDtypeStruct((8, _L_PAD), jnp.int32),
        scratch_shapes=[pltpu.VMEM((_C_PAD, _L_PAD), jnp.float32)],
    )(logits_t, bt[0], bt[1], bt[2], bt[3])
    return labels8[0]


_B_PER_W = _R_PAD // 32


def _sc_bias_body(subj_hbm, obj_hbm, labels_hbm, freq_hbm, rel_hbm, out_hbm,
                  idx_s_v, idx_o_v, pair_v, labels_v, rows_v, rel_v, sem):
    wid = lax.axis_index("s") * 2 + lax.axis_index("c")
    base = wid * _B_PER_W
    pltpu.sync_copy(labels_hbm, labels_v)
    pltpu.sync_copy(subj_hbm.at[pl.ds(base, _B_PER_W)], idx_s_v)
    pltpu.sync_copy(obj_hbm.at[pl.ds(base, _B_PER_W)], idx_o_v)
    for i in range(_B_PER_W // 16):
        s16 = plsc.load_gather(labels_v, [idx_s_v[pl.ds(i * 16, 16)]])
        o16 = plsc.load_gather(labels_v, [idx_o_v[pl.ds(i * 16, 16)]])
        pair_v[pl.ds(i * 16, 16)] = s16 * _NUM_OBJ_CLS + o16
    pltpu.async_copy(freq_hbm.at[pair_v], rows_v, sem).wait()
    pltpu.sync_copy(rel_hbm.at[pl.ds(base, _B_PER_W)], rel_v)

    def addbody(i, _):
        r = i // (_D_PAD // 16)
        c = (i % (_D_PAD // 16)) * 16
        rows_v[r, pl.ds(c, 16)] = (rows_v[r, pl.ds(c, 16)]
                                   + rel_v[r, pl.ds(c, 16)])
        return 0
    lax.fori_loop(0, _B_PER_W * (_D_PAD // 16), addbody, 0)
    pltpu.sync_copy(rows_v, out_hbm.at[pl.ds(base, _B_PER_W)])


def _sc_bias(subj, obj, labels, freq_pad, rel_pad):
    mesh = plsc.VectorSubcoreMesh(core_axis_name="c", subcore_axis_name="s")
    fn = functools.partial(
        pl.kernel,
        mesh=mesh,
        compiler_params=pltpu.CompilerParams(needs_layout_passes=False,
                                             use_tc_tiling_on_sc=False),
        out_type=jax.ShapeDtypeStruct((_R_PAD, _D_PAD), jnp.float32),
        scratch_types=[
            pltpu.VMEM((_B_PER_W,), jnp.int32),
            pltpu.VMEM((_B_PER_W,), jnp.int32),
            pltpu.VMEM((_B_PER_W,), jnp.int32),
            pltpu.VMEM((_L_PAD,), jnp.int32),
            pltpu.VMEM((_B_PER_W, _D_PAD), jnp.float32),
            pltpu.VMEM((_B_PER_W, _D_PAD), jnp.float32),
            pltpu.SemaphoreType.DMA,
        ],
    )(_sc_bias_body)
    return fn(subj, obj, labels, freq_pad, rel_pad)


def kernel(roi_features, union_features, rel_pair_idxs, boxes_per_cls,
           W_obj, W_rel, freq_bias):
    obj_logits = _matmul(roi_features, W_obj, block_rows=400)
    rel_logits = _matmul(union_features, W_rel, block_rows=600)
    pred_label = _nms(obj_logits, boxes_per_cls)

    subj = jnp.pad(rel_pair_idxs[:, 0], (0, _R_PAD - _N_REL))
    obj = jnp.pad(rel_pair_idxs[:, 1], (0, _R_PAD - _N_REL))
    freq_pad = jnp.pad(freq_bias, ((0, 0), (0, _D_PAD - _NUM_REL_CLS)))
    rel_pad = jnp.pad(rel_logits,
                      ((0, _R_PAD - _N_REL), (0, _D_PAD - _NUM_REL_CLS)))
    out = _sc_bias(subj, obj, pred_label, freq_pad, rel_pad)
    return out[:_N_REL, :_NUM_REL_CLS]


# speedup vs baseline: 16.6901x; 1.0360x over previous
import functools

import jax
import jax.numpy as jnp
from jax import lax
from jax.experimental import pallas as pl
from jax.experimental.pallas import tpu as pltpu
from jax.experimental.pallas import tpu_sc as plsc

_NUM_OBJ_CLS = 151
_NUM_REL_CLS = 51
_N_OBJ = 2000
_N_REL = 6000
_IN_CH = 4096
_NMS_THRESH = 0.5

_C_PAD = 152
_L_PAD = 2048
_R_PAD = 6144
_D_PAD = 64
_NEG = -1e30


def _mm_kernel(a_ref, w_ref, o_ref):
    o_ref[...] = jnp.dot(a_ref[...], w_ref[...],
                         preferred_element_type=jnp.float32)


def _matmul(a, w, block_rows):
    n, k = a.shape
    _, m = w.shape
    grid = n // block_rows
    return pl.pallas_call(
        _mm_kernel,
        grid=(grid,),
        in_specs=[
            pl.BlockSpec((block_rows, k), lambda i: (i, 0)),
            pl.BlockSpec((k, m), lambda i: (0, 0)),
        ],
        out_specs=pl.BlockSpec((block_rows, m), lambda i: (i, 0)),
        out_shape=jax.ShapeDtypeStruct((n, m), jnp.float32),
    )(a, w)


def _nms_kernel(logits_t_ref, x1_ref, y1_ref, x2_ref, y2_ref,
                labels_ref, probs_ref):
    cls_iota = lax.broadcasted_iota(jnp.int32, (_C_PAD, _L_PAD), 0)
    lane_iota = lax.broadcasted_iota(jnp.int32, (1, _L_PAD), 1)

    x = logits_t_ref[...]
    m0 = jnp.max(x, axis=0, keepdims=True)
    e = jnp.exp(x - m0)
    s = jnp.sum(e, axis=0, keepdims=True)
    p = e / s
    p = jnp.where(cls_iota == 0, 0.0, p)
    p = jnp.where(cls_iota >= _NUM_OBJ_CLS, 0.0, p)
    probs_ref[...] = p

    active0 = (lane_iota < _N_OBJ).astype(jnp.float32)
    labels0 = jnp.zeros((1, _L_PAD), jnp.int32)
    row_max0 = jnp.max(p, axis=0, keepdims=True)
    row_arg0 = jnp.min(jnp.where(p == row_max0, cls_iota, _C_PAD), axis=0,
                       keepdims=True)

    def body(_, carry):
        active, labels, row_max, row_arg = carry
        masked = jnp.where(active > 0.0, row_max, -1.0)
        m = jnp.max(masked)
        box = jnp.min(jnp.where(masked == m, lane_iota, _L_PAD))
        is_box = lane_iota == box
        cls = jnp.min(jnp.where(is_box, row_arg, _C_PAD))

        x1 = x1_ref[pl.ds(cls, 1), :]
        y1 = y1_ref[pl.ds(cls, 1), :]
        x2 = x2_ref[pl.ds(cls, 1), :]
        y2 = y2_ref[pl.ds(cls, 1), :]
        sx1 = jnp.max(jnp.where(is_box, x1, _NEG))
        sy1 = jnp.max(jnp.where(is_box, y1, _NEG))
        sx2 = jnp.max(jnp.where(is_box, x2, _NEG))
        sy2 = jnp.max(jnp.where(is_box, y2, _NEG))

        ix = jnp.maximum(jnp.minimum(sx2, x2) - jnp.maximum(sx1, x1), 0.0)
        iy = jnp.maximum(jnp.minimum(sy2, y2) - jnp.maximum(sy1, y1), 0.0)
        inter = ix * iy
        a1 = jnp.maximum(sx2 - sx1, 0.0) * jnp.maximum(sy2 - sy1, 0.0)
        a2 = jnp.maximum(x2 - x1, 0.0) * jnp.maximum(y2 - y1, 0.0)
        iou = inter / (a1 + a2 - inter + 1e-9)
        overlap = iou > _NMS_THRESH

        col = probs_ref[pl.ds(cls, 1), :]
        probs_ref[pl.ds(cls, 1), :] = jnp.where(overlap, 0.0, col)

        labels = jnp.where(is_box, cls, labels)
        active = jnp.where(is_box, 0.0, active)

        invalid = (overlap & (active > 0.0)
                   & (col >= row_max) & (row_max > 0.0))
        trigger = jnp.max(jnp.where(invalid, 1.0, 0.0))

        def recompute(_):
            probs = probs_ref[...]
            rm = jnp.max(probs, axis=0, keepdims=True)
            ra = jnp.min(jnp.where(probs == rm, cls_iota, _C_PAD), axis=0,
                         keepdims=True)
            return rm, ra

        row_max, row_arg = lax.cond(trigger > 0.0, recompute,
                                    lambda _: (row_max, row_arg), 0)
        return active, labels, row_max, row_arg

    _, labels, _, _ = lax.fori_loop(
        0, _N_OBJ, body, (active0, labels0, row_max0, row_arg0))
    labels_ref[...] = jnp.broadcast_to(labels, (8, _L_PAD))


def _nms(obj_logits, boxes_per_cls):
    logits_t = jnp.pad(obj_logits.T,
                       ((0, _C_PAD - _NUM_OBJ_CLS), (0, _L_PAD - _N_OBJ)),
                       constant_values=_NEG)
    bt = jnp.pad(boxes_per_cls.transpose(2, 1, 0),
                 ((0, 0), (0, _C_PAD - _NUM_OBJ_CLS), (0, _L_PAD - _N_OBJ)))
    labels8 = pl.pallas_call(
        _nms_kernel,
        out_shape=jax.ShapeDtypeStruct((8, _L_PAD), jnp.int32),
        scratch_shapes=[pltpu.VMEM((_C_PAD, _L_PAD), jnp.float32)],
    )(logits_t, bt[0], bt[1], bt[2], bt[3])
    return labels8[0]


_B_PER_W = _R_PAD // 32


def _sc_bias_body(subj_hbm, obj_hbm, labels_hbm, freq_hbm, rel_hbm, out_hbm,
                  idx_s_v, idx_o_v, pair_v, labels_v, rows_v, rel_v, sem):
    wid = lax.axis_index("s") * 2 + lax.axis_index("c")
    base = wid * _B_PER_W
    pltpu.sync_copy(labels_hbm, labels_v)
    pltpu.sync_copy(subj_hbm.at[pl.ds(base, _B_PER_W)], idx_s_v)
    pltpu.sync_copy(obj_hbm.at[pl.ds(base, _B_PER_W)], idx_o_v)
    for i in range(_B_PER_W // 16):
        s16 = plsc.load_gather(labels_v, [idx_s_v[pl.ds(i * 16, 16)]])
        o16 = plsc.load_gather(labels_v, [idx_o_v[pl.ds(i * 16, 16)]])
        pair_v[pl.ds(i * 16, 16)] = s16 * _NUM_OBJ_CLS + o16
    pltpu.async_copy(freq_hbm.at[pair_v], rows_v, sem).wait()
    pltpu.sync_copy(rel_hbm.at[pl.ds(base, _B_PER_W)], rel_v)

    def addbody(i, _):
        r = i // (_D_PAD // 16)
        c = (i % (_D_PAD // 16)) * 16
        rows_v[r, pl.ds(c, 16)] = (rows_v[r, pl.ds(c, 16)]
                                   + rel_v[r, pl.ds(c, 16)])
        return 0
    lax.fori_loop(0, _B_PER_W * (_D_PAD // 16), addbody, 0)
    pltpu.sync_copy(rows_v, out_hbm.at[pl.ds(base, _B_PER_W)])


def _sc_bias(subj, obj, labels, freq_pad, rel_pad):
    mesh = plsc.VectorSubcoreMesh(core_axis_name="c", subcore_axis_name="s")
    fn = functools.partial(
        pl.kernel,
        mesh=mesh,
        compiler_params=pltpu.CompilerParams(needs_layout_passes=False,
                                             use_tc_tiling_on_sc=False),
        out_type=jax.ShapeDtypeStruct((_R_PAD, _D_PAD), jnp.float32),
        scratch_types=[
            pltpu.VMEM((_B_PER_W,), jnp.int32),
            pltpu.VMEM((_B_PER_W,), jnp.int32),
            pltpu.VMEM((_B_PER_W,), jnp.int32),
            pltpu.VMEM((_L_PAD,), jnp.int32),
            pltpu.VMEM((_B_PER_W, _D_PAD), jnp.float32),
            pltpu.VMEM((_B_PER_W, _D_PAD), jnp.float32),
            pltpu.SemaphoreType.DMA,
        ],
    )(_sc_bias_body)
    return fn(subj, obj, labels, freq_pad, rel_pad)


def kernel(roi_features, union_features, rel_pair_idxs, boxes_per_cls,
           W_obj, W_rel, freq_bias):
    obj_logits = _matmul(roi_features, W_obj, block_rows=400)
    rel_logits = _matmul(union_features, W_rel, block_rows=600)
    pred_label = _nms(obj_logits, boxes_per_cls)

    subj = jnp.pad(rel_pair_idxs[:, 0], (0, _R_PAD - _N_REL))
    obj = jnp.pad(rel_pair_idxs[:, 1], (0, _R_PAD - _N_REL))
    freq_pad = jnp.pad(freq_bias, ((0, 0), (0, _D_PAD - _NUM_REL_CLS)))
    rel_pad = jnp.pad(rel_logits,
                      ((0, _R_PAD - _N_REL), (0, _D_PAD - _NUM_REL_CLS)))
    out = _sc_bias(subj, obj, pred_label, freq_pad, rel_pad)
    return out[:_N_REL, :_NUM_REL_CLS]


# speedup vs baseline: 22.1380x; 1.3264x over previous
import functools

import jax
import jax.numpy as jnp
from jax import lax
from jax.experimental import pallas as pl
from jax.experimental.pallas import tpu as pltpu
from jax.experimental.pallas import tpu_sc as plsc

_NUM_OBJ_CLS = 151
_NUM_REL_CLS = 51
_N_OBJ = 2000
_N_REL = 6000
_IN_CH = 4096
_NMS_THRESH = 0.5

_C_PAD = 152
_L_PAD = 2048
_R_PAD = 6144
_D_PAD = 64
_NEG = -1e30


def _mm_kernel(a_ref, w_ref, o_ref):
    o_ref[...] = jnp.dot(a_ref[...], w_ref[...],
                         preferred_element_type=jnp.float32)


def _matmul(a, w, block_rows):
    n, k = a.shape
    _, m = w.shape
    grid = n // block_rows
    return pl.pallas_call(
        _mm_kernel,
        grid=(grid,),
        in_specs=[
            pl.BlockSpec((block_rows, k), lambda i: (i, 0)),
            pl.BlockSpec((k, m), lambda i: (0, 0)),
        ],
        out_specs=pl.BlockSpec((block_rows, m), lambda i: (i, 0)),
        out_shape=jax.ShapeDtypeStruct((n, m), jnp.float32),
    )(a, w)


def _nms_kernel(logits_t_ref, x1_ref, y1_ref, x2_ref, y2_ref,
                labels_ref, probs_ref, rm_ref, ra_ref):
    cls_iota = lax.broadcasted_iota(jnp.int32, (_C_PAD, _L_PAD), 0)
    lane_iota = lax.broadcasted_iota(jnp.int32, (1, _L_PAD), 1)

    x = logits_t_ref[...]
    m0 = jnp.max(x, axis=0, keepdims=True)
    e = jnp.exp(x - m0)
    s = jnp.sum(e, axis=0, keepdims=True)
    p = e / s
    p = jnp.where(cls_iota == 0, 0.0, p)
    p = jnp.where(cls_iota >= _NUM_OBJ_CLS, 0.0, p)
    probs_ref[...] = p

    row_max0 = jnp.max(p, axis=0, keepdims=True)
    rm0 = jnp.where(lane_iota < _N_OBJ, row_max0, -1.0)
    rm_ref[...] = rm0
    ra0 = jnp.min(jnp.where(p == row_max0, cls_iota, _C_PAD), axis=0,
                  keepdims=True)
    ra_ref[...] = ra0
    labels_ref[...] = jnp.zeros((8, _L_PAD), jnp.int32)

    def pick(rm, ra):
        m = jnp.max(rm)
        box = jnp.min(jnp.where(rm == m, lane_iota, _L_PAD))
        cls = jnp.min(jnp.where(lane_iota == box, ra, _C_PAD))
        return box, cls

    def body(_, bc):
        box, cls = bc
        is_box = lane_iota == box

        x1 = x1_ref[pl.ds(cls, 1), :]
        y1 = y1_ref[pl.ds(cls, 1), :]
        x2 = x2_ref[pl.ds(cls, 1), :]
        y2 = y2_ref[pl.ds(cls, 1), :]
        sx1 = jnp.max(jnp.where(is_box, x1, _NEG))
        sy1 = jnp.max(jnp.where(is_box, y1, _NEG))
        sx2 = jnp.max(jnp.where(is_box, x2, _NEG))
        sy2 = jnp.max(jnp.where(is_box, y2, _NEG))

        ix = jnp.maximum(jnp.minimum(sx2, x2) - jnp.maximum(sx1, x1), 0.0)
        iy = jnp.maximum(jnp.minimum(sy2, y2) - jnp.maximum(sy1, y1), 0.0)
        inter = ix * iy
        a1 = jnp.maximum(sx2 - sx1, 0.0) * jnp.maximum(sy2 - sy1, 0.0)
        a2 = jnp.maximum(x2 - x1, 0.0) * jnp.maximum(y2 - y1, 0.0)
        iou = inter / (a1 + a2 - inter + 1e-9)
        overlap = iou > _NMS_THRESH

        col = probs_ref[pl.ds(cls, 1), :]
        probs_ref[pl.ds(cls, 1), :] = jnp.where(overlap, 0.0, col)

        lab = labels_ref[pl.ds(0, 1), :]
        labels_ref[pl.ds(0, 1), :] = jnp.where(is_box, cls, lab)
        rm1 = jnp.where(is_box, -1.0, rm_ref[...])
        rm_ref[...] = rm1

        nbox, ncls = pick(rm1, ra_ref[...])

        invalid = (overlap & (rm1 > 0.0) & (col >= rm1))
        trigger = jnp.max(jnp.where(invalid, 1.0, 0.0))

        def redo(_):
            probs = probs_ref[...]
            rm = jnp.max(probs, axis=0, keepdims=True)
            ra = jnp.min(jnp.where(probs == rm, cls_iota, _C_PAD), axis=0,
                         keepdims=True)
            rm = jnp.where(rm1 < 0.0, -1.0, rm)
            rm_ref[...] = rm
            ra_ref[...] = ra
            return pick(rm, ra)

        return lax.cond(trigger > 0.0, redo, lambda _: (nbox, ncls), 0)

    lax.fori_loop(0, _N_OBJ, body, pick(rm0, ra0))


def _nms(obj_logits, boxes_per_cls):
    logits_t = jnp.pad(obj_logits.T,
                       ((0, _C_PAD - _NUM_OBJ_CLS), (0, _L_PAD - _N_OBJ)),
                       constant_values=_NEG)
    bt = jnp.pad(boxes_per_cls.transpose(2, 1, 0),
                 ((0, 0), (0, _C_PAD - _NUM_OBJ_CLS), (0, _L_PAD - _N_OBJ)))
    labels8 = pl.pallas_call(
        _nms_kernel,
        out_shape=jax.ShapeDtypeStruct((8, _L_PAD), jnp.int32),
        scratch_shapes=[pltpu.VMEM((_C_PAD, _L_PAD), jnp.float32),
                        pltpu.VMEM((1, _L_PAD), jnp.float32),
                        pltpu.VMEM((1, _L_PAD), jnp.int32)],
    )(logits_t, bt[0], bt[1], bt[2], bt[3])
    return labels8[0]


_B_PER_W = _R_PAD // 32


def _sc_bias_body(subj_hbm, obj_hbm, labels_hbm, freq_hbm, rel_hbm, out_hbm,
                  idx_s_v, idx_o_v, pair_v, labels_v, rows_v, rel_v, sem):
    wid = lax.axis_index("s") * 2 + lax.axis_index("c")
    base = wid * _B_PER_W
    pltpu.sync_copy(labels_hbm, labels_v)
    pltpu.sync_copy(subj_hbm.at[pl.ds(base, _B_PER_W)], idx_s_v)
    pltpu.sync_copy(obj_hbm.at[pl.ds(base, _B_PER_W)], idx_o_v)
    for i in range(_B_PER_W // 16):
        s16 = plsc.load_gather(labels_v, [idx_s_v[pl.ds(i * 16, 16)]])
        o16 = plsc.load_gather(labels_v, [idx_o_v[pl.ds(i * 16, 16)]])
        pair_v[pl.ds(i * 16, 16)] = s16 * _NUM_OBJ_CLS + o16
    pltpu.async_copy(freq_hbm.at[pair_v], rows_v, sem).wait()
    pltpu.sync_copy(rel_hbm.at[pl.ds(base, _B_PER_W)], rel_v)

    def addbody(i, _):
        r = i // (_D_PAD // 16)
        c = (i % (_D_PAD // 16)) * 16
        rows_v[r, pl.ds(c, 16)] = (rows_v[r, pl.ds(c, 16)]
                                   + rel_v[r, pl.ds(c, 16)])
        return 0
    lax.fori_loop(0, _B_PER_W * (_D_PAD // 16), addbody, 0)
    pltpu.sync_copy(rows_v, out_hbm.at[pl.ds(base, _B_PER_W)])


def _sc_bias(subj, obj, labels, freq_pad, rel_pad):
    mesh = plsc.VectorSubcoreMesh(core_axis_name="c", subcore_axis_name="s")
    fn = functools.partial(
        pl.kernel,
        mesh=mesh,
        compiler_params=pltpu.CompilerParams(needs_layout_passes=False,
                                             use_tc_tiling_on_sc=False),
        out_type=jax.ShapeDtypeStruct((_R_PAD, _D_PAD), jnp.float32),
        scratch_types=[
            pltpu.VMEM((_B_PER_W,), jnp.int32),
            pltpu.VMEM((_B_PER_W,), jnp.int32),
            pltpu.VMEM((_B_PER_W,), jnp.int32),
            pltpu.VMEM((_L_PAD,), jnp.int32),
            pltpu.VMEM((_B_PER_W, _D_PAD), jnp.float32),
            pltpu.VMEM((_B_PER_W, _D_PAD), jnp.float32),
            pltpu.SemaphoreType.DMA,
        ],
    )(_sc_bias_body)
    return fn(subj, obj, labels, freq_pad, rel_pad)


def kernel(roi_features, union_features, rel_pair_idxs, boxes_per_cls,
           W_obj, W_rel, freq_bias):
    obj_logits = _matmul(roi_features, W_obj, block_rows=400)
    rel_logits = _matmul(union_features, W_rel, block_rows=600)
    pred_label = _nms(obj_logits, boxes_per_cls)

    subj = jnp.pad(rel_pair_idxs[:, 0], (0, _R_PAD - _N_REL))
    obj = jnp.pad(rel_pair_idxs[:, 1], (0, _R_PAD - _N_REL))
    freq_pad = jnp.pad(freq_bias, ((0, 0), (0, _D_PAD - _NUM_REL_CLS)))
    rel_pad = jnp.pad(rel_logits,
                      ((0, _R_PAD - _N_REL), (0, _D_PAD - _NUM_REL_CLS)))
    out = _sc_bias(subj, obj, pred_label, freq_pad, rel_pad)
    return out[:_N_REL, :_NUM_REL_CLS]


# speedup vs baseline: 36.7385x; 1.6595x over previous
import functools

import jax
import jax.numpy as jnp
from jax import lax
from jax.experimental import pallas as pl
from jax.experimental.pallas import tpu as pltpu
from jax.experimental.pallas import tpu_sc as plsc

_NUM_OBJ_CLS = 151
_NUM_REL_CLS = 51
_N_OBJ = 2000
_N_REL = 6000
_IN_CH = 4096
_NMS_THRESH = 0.5

_C_PAD = 152
_L_PAD = 2048
_R_PAD = 6144
_D_PAD = 64
_NEG = -1e30


def _mm_kernel(a_ref, w_ref, o_ref):
    o_ref[...] = jnp.dot(a_ref[...], w_ref[...],
                         preferred_element_type=jnp.float32)


def _matmul(a, w, block_rows):
    n, k = a.shape
    _, m = w.shape
    grid = n // block_rows
    return pl.pallas_call(
        _mm_kernel,
        grid=(grid,),
        in_specs=[
            pl.BlockSpec((block_rows, k), lambda i: (i, 0)),
            pl.BlockSpec((k, m), lambda i: (0, 0)),
        ],
        out_specs=pl.BlockSpec((block_rows, m), lambda i: (i, 0)),
        out_shape=jax.ShapeDtypeStruct((n, m), jnp.float32),
    )(a, w)


_BIG = 3.0e7


def _nms_kernel(logits_t_ref, x1_ref, y1_ref, x2_ref, y2_ref,
                labels_ref, probs_ref, rm_ref, ra4_ref, stale_ref):
    cls_iota = lax.broadcasted_iota(jnp.int32, (_C_PAD, _L_PAD), 0)
    lane_iota = lax.broadcasted_iota(jnp.int32, (1, _L_PAD), 1)
    lane_key = lane_iota.astype(jnp.float32) * 1024.0

    x = logits_t_ref[...]
    m0 = jnp.max(x, axis=0, keepdims=True)
    e = jnp.exp(x - m0)
    s = jnp.sum(e, axis=0, keepdims=True)
    p = e / s
    p = jnp.where(cls_iota == 0, 0.0, p)
    p = jnp.where(cls_iota >= _NUM_OBJ_CLS, 0.0, p)
    probs_ref[...] = p

    row_max0 = jnp.max(p, axis=0, keepdims=True)
    rm0 = jnp.where(lane_iota < _N_OBJ, row_max0, -1.0)
    rm_ref[...] = rm0
    ra4_0 = jnp.min(jnp.where(p == row_max0, cls_iota, _C_PAD), axis=0,
                    keepdims=True).astype(jnp.float32) * 4.0
    ra4_ref[...] = ra4_0
    stale_ref[...] = jnp.zeros((1, _L_PAD), jnp.float32)
    labels_ref[...] = jnp.zeros((8, _L_PAD), jnp.int32)

    def pick(rm, key_extra):
        m = jnp.max(rm)
        r = jnp.min(jnp.where(rm == m, key_extra, _BIG)).astype(jnp.int32)
        box = r >> 10
        rem = r - (box << 10)
        return box, rem >> 2, rem & 3

    def fresh_pick(rm1):
        probs = probs_ref[...]
        rm = jnp.max(probs, axis=0, keepdims=True)
        ra4 = jnp.min(jnp.where(probs == rm, cls_iota, _C_PAD), axis=0,
                      keepdims=True).astype(jnp.float32) * 4.0
        rm = jnp.where(rm1 < 0.0, -1.0, rm)
        rm_ref[...] = rm
        ra4_ref[...] = ra4
        stale_ref[...] = jnp.zeros((1, _L_PAD), jnp.float32)
        box, cls, _ = pick(rm, lane_key + ra4)
        return box, cls

    def body(_, bc):
        box, cls = bc
        is_box = lane_iota == box

        x1 = x1_ref[pl.ds(cls, 1), :]
        y1 = y1_ref[pl.ds(cls, 1), :]
        x2 = x2_ref[pl.ds(cls, 1), :]
        y2 = y2_ref[pl.ds(cls, 1), :]
        stack = jnp.concatenate(
            [jnp.where(is_box, x1, _NEG), jnp.where(is_box, y1, _NEG),
             jnp.where(is_box, x2, _NEG), jnp.where(is_box, y2, _NEG)],
            axis=0)
        sel = jnp.max(stack, axis=1, keepdims=True)
        sx1 = sel[0, 0]
        sy1 = sel[1, 0]
        sx2 = sel[2, 0]
        sy2 = sel[3, 0]

        ix = jnp.maximum(jnp.minimum(sx2, x2) - jnp.maximum(sx1, x1), 0.0)
        iy = jnp.maximum(jnp.minimum(sy2, y2) - jnp.maximum(sy1, y1), 0.0)
        inter = ix * iy
        a1 = jnp.maximum(sx2 - sx1, 0.0) * jnp.maximum(sy2 - sy1, 0.0)
        a2 = jnp.maximum(x2 - x1, 0.0) * jnp.maximum(y2 - y1, 0.0)
        iou = inter / (a1 + a2 - inter + 1e-9)
        overlap = iou > _NMS_THRESH

        col = probs_ref[pl.ds(cls, 1), :]
        probs_ref[pl.ds(cls, 1), :] = jnp.where(overlap, 0.0, col)

        lab = labels_ref[pl.ds(0, 1), :]
        labels_ref[pl.ds(0, 1), :] = jnp.where(is_box, cls, lab)
        rm1 = jnp.where(is_box, -1.0, rm_ref[...])
        rm_ref[...] = rm1

        invalid = (overlap & (rm1 > 0.0) & (col >= rm1))
        stale = jnp.where(invalid, 1.0, stale_ref[...])
        stale_ref[...] = stale

        nbox, ncls, nstale = pick(rm1, lane_key + ra4_ref[...] + stale)
        return lax.cond(nstale > 0, lambda _: fresh_pick(rm1),
                        lambda _: (nbox, ncls), 0)

    b0, c0, _ = pick(rm0, lane_key + ra4_0)
    lax.fori_loop(0, _N_OBJ, body, (b0, c0))


def _nms(obj_logits, boxes_per_cls):
    logits_t = jnp.pad(obj_logits.T,
                       ((0, _C_PAD - _NUM_OBJ_CLS), (0, _L_PAD - _N_OBJ)),
                       constant_values=_NEG)
    bt = jnp.pad(boxes_per_cls.transpose(2, 1, 0),
                 ((0, 0), (0, _C_PAD - _NUM_OBJ_CLS), (0, _L_PAD - _N_OBJ)))
    labels8 = pl.pallas_call(
        _nms_kernel,
        out_shape=jax.ShapeDtypeStruct((8, _L_PAD), jnp.int32),
        scratch_shapes=[pltpu.VMEM((_C_PAD, _L_PAD), jnp.float32),
                        pltpu.VMEM((1, _L_PAD), jnp.float32),
                        pltpu.VMEM((1, _L_PAD), jnp.float32),
                        pltpu.VMEM((1, _L_PAD), jnp.float32)],
    )(logits_t, bt[0], bt[1], bt[2], bt[3])
    return labels8[0]


_B_PER_W = _R_PAD // 32


def _sc_bias_body(subj_hbm, obj_hbm, labels_hbm, freq_hbm, rel_hbm, out_hbm,
                  idx_s_v, idx_o_v, pair_v, labels_v, rows_v, rel_v, sem):
    wid = lax.axis_index("s") * 2 + lax.axis_index("c")
    base = wid * _B_PER_W
    pltpu.sync_copy(labels_hbm, labels_v)
    pltpu.sync_copy(subj_hbm.at[pl.ds(base, _B_PER_W)], idx_s_v)
    pltpu.sync_copy(obj_hbm.at[pl.ds(base, _B_PER_W)], idx_o_v)
    for i in range(_B_PER_W // 16):
        s16 = plsc.load_gather(labels_v, [idx_s_v[pl.ds(i * 16, 16)]])
        o16 = plsc.load_gather(labels_v, [idx_o_v[pl.ds(i * 16, 16)]])
        pair_v[pl.ds(i * 16, 16)] = s16 * _NUM_OBJ_CLS + o16
    pltpu.async_copy(freq_hbm.at[pair_v], rows_v, sem).wait()
    pltpu.sync_copy(rel_hbm.at[pl.ds(base, _B_PER_W)], rel_v)

    def addbody(i, _):
        r = i // (_D_PAD // 16)
        c = (i % (_D_PAD // 16)) * 16
        rows_v[r, pl.ds(c, 16)] = (rows_v[r, pl.ds(c, 16)]
                                   + rel_v[r, pl.ds(c, 16)])
        return 0
    lax.fori_loop(0, _B_PER_W * (_D_PAD // 16), addbody, 0)
    pltpu.sync_copy(rows_v, out_hbm.at[pl.ds(base, _B_PER_W)])


def _sc_bias(subj, obj, labels, freq_pad, rel_pad):
    mesh = plsc.VectorSubcoreMesh(core_axis_name="c", subcore_axis_name="s")
    fn = functools.partial(
        pl.kernel,
        mesh=mesh,
        compiler_params=pltpu.CompilerParams(needs_layout_passes=False,
                                             use_tc_tiling_on_sc=False),
        out_type=jax.ShapeDtypeStruct((_R_PAD, _D_PAD), jnp.float32),
        scratch_types=[
            pltpu.VMEM((_B_PER_W,), jnp.int32),
            pltpu.VMEM((_B_PER_W,), jnp.int32),
            pltpu.VMEM((_B_PER_W,), jnp.int32),
            pltpu.VMEM((_L_PAD,), jnp.int32),
            pltpu.VMEM((_B_PER_W, _D_PAD), jnp.float32),
            pltpu.VMEM((_B_PER_W, _D_PAD), jnp.float32),
            pltpu.SemaphoreType.DMA,
        ],
    )(_sc_bias_body)
    return fn(subj, obj, labels, freq_pad, rel_pad)


def kernel(roi_features, union_features, rel_pair_idxs, boxes_per_cls,
           W_obj, W_rel, freq_bias):
    obj_logits = _matmul(roi_features, W_obj, block_rows=400)
    rel_logits = _matmul(union_features, W_rel, block_rows=600)
    pred_label = _nms(obj_logits, boxes_per_cls)

    subj = jnp.pad(rel_pair_idxs[:, 0], (0, _R_PAD - _N_REL))
    obj = jnp.pad(rel_pair_idxs[:, 1], (0, _R_PAD - _N_REL))
    freq_pad = jnp.pad(freq_bias, ((0, 0), (0, _D_PAD - _NUM_REL_CLS)))
    rel_pad = jnp.pad(rel_logits,
                      ((0, _R_PAD - _N_REL), (0, _D_PAD - _NUM_REL_CLS)))
    out = _sc_bias(subj, obj, pred_label, freq_pad, rel_pad)
    return out[:_N_REL, :_NUM_REL_CLS]


# speedup vs baseline: 45.3967x; 1.2357x over previous
import functools

import jax
import jax.numpy as jnp
from jax import lax
from jax.experimental import pallas as pl
from jax.experimental.pallas import tpu as pltpu
from jax.experimental.pallas import tpu_sc as plsc

_NUM_OBJ_CLS = 151
_NUM_REL_CLS = 51
_N_OBJ = 2000
_N_REL = 6000
_IN_CH = 4096
_NMS_THRESH = 0.5

_C_PAD = 152
_L_PAD = 2048
_R_PAD = 6144
_D_PAD = 64
_NEG = -1e30


def _mm_kernel(a_ref, w_ref, o_ref):
    o_ref[...] = jnp.dot(a_ref[...], w_ref[...],
                         preferred_element_type=jnp.float32)


def _matmul(a, w, block_rows):
    n, k = a.shape
    _, m = w.shape
    grid = n // block_rows
    return pl.pallas_call(
        _mm_kernel,
        grid=(grid,),
        in_specs=[
            pl.BlockSpec((block_rows, k), lambda i: (i, 0)),
            pl.BlockSpec((k, m), lambda i: (0, 0)),
        ],
        out_specs=pl.BlockSpec((block_rows, m), lambda i: (i, 0)),
        out_shape=jax.ShapeDtypeStruct((n, m), jnp.float32),
    )(a, w)


_BIG = 3.0e7
_SL = 16
_LN = 128


def _nms_kernel(logits_t_ref, x1_ref, y1_ref, x2_ref, y2_ref,
                labels_ref, probs_ref, rm_ref, ra4_ref, stale_ref):
    cls_iota = lax.broadcasted_iota(jnp.int32, (_C_PAD, _SL, _LN), 0)
    box_iota = (lax.broadcasted_iota(jnp.int32, (_SL, _LN), 0) * _LN
                + lax.broadcasted_iota(jnp.int32, (_SL, _LN), 1))
    lane_key = box_iota.astype(jnp.float32) * 1024.0

    x = logits_t_ref[...]
    m0 = jnp.max(x, axis=0, keepdims=True)
    e = jnp.exp(x - m0)
    s = jnp.sum(e, axis=0, keepdims=True)
    p = e / s
    p = jnp.where(cls_iota == 0, 0.0, p)
    p = jnp.where(cls_iota >= _NUM_OBJ_CLS, 0.0, p)
    probs_ref[...] = p

    row_max0 = jnp.max(p, axis=0)
    rm0 = jnp.where(box_iota < _N_OBJ, row_max0, -1.0)
    rm_ref[...] = rm0
    ra4_0 = jnp.min(jnp.where(p == row_max0, cls_iota, _C_PAD),
                    axis=0).astype(jnp.float32) * 4.0
    ra4_ref[...] = ra4_0
    stale_ref[...] = jnp.zeros((_SL, _LN), jnp.float32)
    labels_ref[...] = jnp.zeros((_SL, _LN), jnp.int32)

    def pick(rm, key_extra):
        m = jnp.max(rm)
        r = jnp.min(jnp.where(rm == m, key_extra, _BIG)).astype(jnp.int32)
        box = r >> 10
        rem = r - (box << 10)
        return box, rem >> 2, rem & 3

    def fresh_pick(rm1):
        probs = probs_ref[...]
        rm = jnp.max(probs, axis=0)
        ra4 = jnp.min(jnp.where(probs == rm, cls_iota, _C_PAD),
                      axis=0).astype(jnp.float32) * 4.0
        rm = jnp.where(rm1 < 0.0, -1.0, rm)
        rm_ref[...] = rm
        ra4_ref[...] = ra4
        stale_ref[...] = jnp.zeros((_SL, _LN), jnp.float32)
        box, cls, _ = pick(rm, lane_key + ra4)
        return box, cls

    def body(_, bc):
        box, cls = bc
        is_box = box_iota == box

        x1 = x1_ref[pl.ds(cls, 1), :, :][0]
        y1 = y1_ref[pl.ds(cls, 1), :, :][0]
        x2 = x2_ref[pl.ds(cls, 1), :, :][0]
        y2 = y2_ref[pl.ds(cls, 1), :, :][0]
        stack = jnp.concatenate(
            [jnp.where(is_box, x1, _NEG)[None], jnp.where(is_box, y1, _NEG)[None],
             jnp.where(is_box, x2, _NEG)[None], jnp.where(is_box, y2, _NEG)[None]],
            axis=0)
        sel = jnp.max(jnp.max(stack, axis=1), axis=1, keepdims=True)
        sx1 = sel[0, 0]
        sy1 = sel[1, 0]
        sx2 = sel[2, 0]
        sy2 = sel[3, 0]

        ix = jnp.maximum(jnp.minimum(sx2, x2) - jnp.maximum(sx1, x1), 0.0)
        iy = jnp.maximum(jnp.minimum(sy2, y2) - jnp.maximum(sy1, y1), 0.0)
        inter = ix * iy
        a1 = jnp.maximum(sx2 - sx1, 0.0) * jnp.maximum(sy2 - sy1, 0.0)
        a2 = jnp.maximum(x2 - x1, 0.0) * jnp.maximum(y2 - y1, 0.0)
        iou = inter / (a1 + a2 - inter + 1e-9)
        overlap = iou > _NMS_THRESH

        col = probs_ref[pl.ds(cls, 1), :, :][0]
        probs_ref[pl.ds(cls, 1), :, :] = jnp.where(overlap, 0.0, col)[None]

        labels_ref[...] = jnp.where(is_box, cls, labels_ref[...])
        rm1 = jnp.where(is_box, -1.0, rm_ref[...])
        rm_ref[...] = rm1

        invalid = (overlap & (rm1 > 0.0) & (col >= rm1))
        stale = jnp.where(invalid, 1.0, stale_ref[...])
        stale_ref[...] = stale

        nbox, ncls, nstale = pick(rm1, lane_key + ra4_ref[...] + stale)
        return lax.cond(nstale > 0, lambda _: fresh_pick(rm1),
                        lambda _: (nbox, ncls), 0)

    b0, c0, _ = pick(rm0, lane_key + ra4_0)
    lax.fori_loop(0, _N_OBJ, body, (b0, c0))


def _nms(obj_logits, boxes_per_cls):
    logits_t = jnp.pad(obj_logits.T,
                       ((0, _C_PAD - _NUM_OBJ_CLS), (0, _L_PAD - _N_OBJ)),
                       constant_values=_NEG).reshape(_C_PAD, _SL, _LN)
    bt = jnp.pad(boxes_per_cls.transpose(2, 1, 0),
                 ((0, 0), (0, _C_PAD - _NUM_OBJ_CLS), (0, _L_PAD - _N_OBJ))
                 ).reshape(4, _C_PAD, _SL, _LN)
    labels = pl.pallas_call(
        _nms_kernel,
        out_shape=jax.ShapeDtypeStruct((_SL, _LN), jnp.int32),
        scratch_shapes=[pltpu.VMEM((_C_PAD, _SL, _LN), jnp.float32),
                        pltpu.VMEM((_SL, _LN), jnp.float32),
                        pltpu.VMEM((_SL, _LN), jnp.float32),
                        pltpu.VMEM((_SL, _LN), jnp.float32)],
    )(logits_t, bt[0], bt[1], bt[2], bt[3])
    return labels.reshape(_L_PAD)


_B_PER_W = _R_PAD // 32


def _sc_bias_body(subj_hbm, obj_hbm, labels_hbm, freq_hbm, rel_hbm, out_hbm,
                  idx_s_v, idx_o_v, pair_v, labels_v, rows_v, rel_v, sem):
    wid = lax.axis_index("s") * 2 + lax.axis_index("c")
    base = wid * _B_PER_W
    pltpu.sync_copy(labels_hbm, labels_v)
    pltpu.sync_copy(subj_hbm.at[pl.ds(base, _B_PER_W)], idx_s_v)
    pltpu.sync_copy(obj_hbm.at[pl.ds(base, _B_PER_W)], idx_o_v)
    for i in range(_B_PER_W // 16):
        s16 = plsc.load_gather(labels_v, [idx_s_v[pl.ds(i * 16, 16)]])
        o16 = plsc.load_gather(labels_v, [idx_o_v[pl.ds(i * 16, 16)]])
        pair_v[pl.ds(i * 16, 16)] = s16 * _NUM_OBJ_CLS + o16
    pltpu.async_copy(freq_hbm.at[pair_v], rows_v, sem).wait()
    pltpu.sync_copy(rel_hbm.at[pl.ds(base, _B_PER_W)], rel_v)

    def addbody(i, _):
        r = i // (_D_PAD // 16)
        c = (i % (_D_PAD // 16)) * 16
        rows_v[r, pl.ds(c, 16)] = (rows_v[r, pl.ds(c, 16)]
                                   + rel_v[r, pl.ds(c, 16)])
        return 0
    lax.fori_loop(0, _B_PER_W * (_D_PAD // 16), addbody, 0)
    pltpu.sync_copy(rows_v, out_hbm.at[pl.ds(base, _B_PER_W)])


def _sc_bias(subj, obj, labels, freq_pad, rel_pad):
    mesh = plsc.VectorSubcoreMesh(core_axis_name="c", subcore_axis_name="s")
    fn = functools.partial(
        pl.kernel,
        mesh=mesh,
        compiler_params=pltpu.CompilerParams(needs_layout_passes=False,
                                             use_tc_tiling_on_sc=False),
        out_type=jax.ShapeDtypeStruct((_R_PAD, _D_PAD), jnp.float32),
        scratch_types=[
            pltpu.VMEM((_B_PER_W,), jnp.int32),
            pltpu.VMEM((_B_PER_W,), jnp.int32),
            pltpu.VMEM((_B_PER_W,), jnp.int32),
            pltpu.VMEM((_L_PAD,), jnp.int32),
            pltpu.VMEM((_B_PER_W, _D_PAD), jnp.float32),
            pltpu.VMEM((_B_PER_W, _D_PAD), jnp.float32),
            pltpu.SemaphoreType.DMA,
        ],
    )(_sc_bias_body)
    return fn(subj, obj, labels, freq_pad, rel_pad)


def kernel(roi_features, union_features, rel_pair_idxs, boxes_per_cls,
           W_obj, W_rel, freq_bias):
    obj_logits = _matmul(roi_features, W_obj, block_rows=400)
    rel_logits = _matmul(union_features, W_rel, block_rows=600)
    pred_label = _nms(obj_logits, boxes_per_cls)

    subj = jnp.pad(rel_pair_idxs[:, 0], (0, _R_PAD - _N_REL))
    obj = jnp.pad(rel_pair_idxs[:, 1], (0, _R_PAD - _N_REL))
    freq_pad = jnp.pad(freq_bias, ((0, 0), (0, _D_PAD - _NUM_REL_CLS)))
    rel_pad = jnp.pad(rel_logits,
                      ((0, _R_PAD - _N_REL), (0, _D_PAD - _NUM_REL_CLS)))
    out = _sc_bias(subj, obj, pred_label, freq_pad, rel_pad)
    return out[:_N_REL, :_NUM_REL_CLS]


# speedup vs baseline: 47.4314x; 1.0448x over previous
import functools

import jax
import jax.numpy as jnp
from jax import lax
from jax.experimental import pallas as pl
from jax.experimental.pallas import tpu as pltpu
from jax.experimental.pallas import tpu_sc as plsc

_NUM_OBJ_CLS = 151
_NUM_REL_CLS = 51
_N_OBJ = 2000
_N_REL = 6000
_IN_CH = 4096
_NMS_THRESH = 0.5

_C_PAD = 152
_L_PAD = 2048
_R_PAD = 6144
_D_PAD = 64
_NEG = -1e30


def _mm_kernel(a_ref, w_ref, o_ref):
    o_ref[...] = jnp.dot(a_ref[...], w_ref[...],
                         preferred_element_type=jnp.float32)


def _matmul(a, w, block_rows):
    n, k = a.shape
    _, m = w.shape
    grid = n // block_rows
    return pl.pallas_call(
        _mm_kernel,
        grid=(grid,),
        in_specs=[
            pl.BlockSpec((block_rows, k), lambda i: (i, 0)),
            pl.BlockSpec((k, m), lambda i: (0, 0)),
        ],
        out_specs=pl.BlockSpec((block_rows, m), lambda i: (i, 0)),
        out_shape=jax.ShapeDtypeStruct((n, m), jnp.float32),
    )(a, w)


_BIG = 3.0e7
_SL = 16
_LN = 128


def _nms_kernel(logits_t_ref, x1_ref, y1_ref, x2_ref, y2_ref,
                labels_ref, probs_ref, rm_ref, ra4_ref, stale_ref):
    cls_iota = lax.broadcasted_iota(jnp.int32, (_C_PAD, _SL, _LN), 0)
    box_iota = (lax.broadcasted_iota(jnp.int32, (_SL, _LN), 0) * _LN
                + lax.broadcasted_iota(jnp.int32, (_SL, _LN), 1))
    lane_key = box_iota.astype(jnp.float32) * 1024.0

    x = logits_t_ref[...]
    m0 = jnp.max(x, axis=0, keepdims=True)
    e = jnp.exp(x - m0)
    s = jnp.sum(e, axis=0, keepdims=True)
    p = e / s
    p = jnp.where(cls_iota == 0, 0.0, p)
    p = jnp.where(cls_iota >= _NUM_OBJ_CLS, 0.0, p)
    probs_ref[...] = p

    row_max0 = jnp.max(p, axis=0)
    rm0 = jnp.where(box_iota < _N_OBJ, row_max0, -1.0)
    rm_ref[...] = rm0
    ra4_0 = jnp.min(jnp.where(p == row_max0, cls_iota, _C_PAD),
                    axis=0).astype(jnp.float32) * 4.0
    ra4_ref[...] = ra4_0
    stale_ref[...] = jnp.zeros((_SL, _LN), jnp.float32)
    labels_ref[...] = jnp.zeros((_SL, _LN), jnp.int32)

    def pick(rm, key_extra):
        m = jnp.max(rm)
        r = jnp.min(jnp.where(rm == m, key_extra, _BIG)).astype(jnp.int32)
        box = r >> 10
        rem = r - (box << 10)
        return box, rem >> 2, rem & 3

    def fresh_pick(rm1):
        probs = probs_ref[...]
        rm = jnp.max(probs, axis=0)
        ra4 = jnp.min(jnp.where(probs == rm, cls_iota, _C_PAD),
                      axis=0).astype(jnp.float32) * 4.0
        rm = jnp.where(rm1 < 0.0, -1.0, rm)
        rm_ref[...] = rm
        ra4_ref[...] = ra4
        stale_ref[...] = jnp.zeros((_SL, _LN), jnp.float32)
        box, cls, _ = pick(rm, lane_key + ra4)
        return box, cls

    def body(_, bc):
        box, cls = bc
        is_box = box_iota == box

        col = probs_ref[pl.ds(cls, 1), :, :][0]
        rm1 = jnp.where(is_box, -1.0, rm_ref[...])
        rm_ref[...] = rm1
        candidate = (col >= rm1) & (rm1 > 0.0)
        stale_prev = stale_ref[...]
        flag = jnp.where(candidate, 1.0, stale_prev)
        nbox, ncls, nstale = pick(rm1, lane_key + ra4_ref[...] + flag)

        x1 = x1_ref[pl.ds(cls, 1), :, :][0]
        y1 = y1_ref[pl.ds(cls, 1), :, :][0]
        x2 = x2_ref[pl.ds(cls, 1), :, :][0]
        y2 = y2_ref[pl.ds(cls, 1), :, :][0]
        stack = jnp.concatenate(
            [jnp.where(is_box, x1, _NEG)[None], jnp.where(is_box, y1, _NEG)[None],
             jnp.where(is_box, x2, _NEG)[None], jnp.where(is_box, y2, _NEG)[None]],
            axis=0)
        sel = jnp.max(jnp.max(stack, axis=1), axis=1, keepdims=True)
        sx1 = sel[0, 0]
        sy1 = sel[1, 0]
        sx2 = sel[2, 0]
        sy2 = sel[3, 0]

        ix = jnp.maximum(jnp.minimum(sx2, x2) - jnp.maximum(sx1, x1), 0.0)
        iy = jnp.maximum(jnp.minimum(sy2, y2) - jnp.maximum(sy1, y1), 0.0)
        inter = ix * iy
        a1 = jnp.maximum(sx2 - sx1, 0.0) * jnp.maximum(sy2 - sy1, 0.0)
        a2 = jnp.maximum(x2 - x1, 0.0) * jnp.maximum(y2 - y1, 0.0)
        iou = inter / (a1 + a2 - inter + 1e-9)
        overlap = iou > _NMS_THRESH

        probs_ref[pl.ds(cls, 1), :, :] = jnp.where(overlap, 0.0, col)[None]
        labels_ref[...] = jnp.where(is_box, cls, labels_ref[...])

        invalid = overlap & candidate
        stale_ref[...] = jnp.where(invalid, 1.0, stale_prev)

        return lax.cond(nstale > 0, lambda _: fresh_pick(rm1),
                        lambda _: (nbox, ncls), 0)

    b0, c0, _ = pick(rm0, lane_key + ra4_0)
    lax.fori_loop(0, _N_OBJ, body, (b0, c0))


def _nms(obj_logits, boxes_per_cls):
    logits_t = jnp.pad(obj_logits.T,
                       ((0, _C_PAD - _NUM_OBJ_CLS), (0, _L_PAD - _N_OBJ)),
                       constant_values=_NEG).reshape(_C_PAD, _SL, _LN)
    bt = jnp.pad(boxes_per_cls.transpose(2, 1, 0),
                 ((0, 0), (0, _C_PAD - _NUM_OBJ_CLS), (0, _L_PAD - _N_OBJ))
                 ).reshape(4, _C_PAD, _SL, _LN)
    labels = pl.pallas_call(
        _nms_kernel,
        out_shape=jax.ShapeDtypeStruct((_SL, _LN), jnp.int32),
        scratch_shapes=[pltpu.VMEM((_C_PAD, _SL, _LN), jnp.float32),
                        pltpu.VMEM((_SL, _LN), jnp.float32),
                        pltpu.VMEM((_SL, _LN), jnp.float32),
                        pltpu.VMEM((_SL, _LN), jnp.float32)],
    )(logits_t, bt[0], bt[1], bt[2], bt[3])
    return labels.reshape(_L_PAD)


_B_PER_W = _R_PAD // 32


def _sc_bias_body(subj_hbm, obj_hbm, labels_hbm, freq_hbm, rel_hbm, out_hbm,
                  idx_s_v, idx_o_v, pair_v, labels_v, rows_v, rel_v, sem):
    wid = lax.axis_index("s") * 2 + lax.axis_index("c")
    base = wid * _B_PER_W
    pltpu.sync_copy(labels_hbm, labels_v)
    pltpu.sync_copy(subj_hbm.at[pl.ds(base, _B_PER_W)], idx_s_v)
    pltpu.sync_copy(obj_hbm.at[pl.ds(base, _B_PER_W)], idx_o_v)
    for i in range(_B_PER_W // 16):
        s16 = plsc.load_gather(labels_v, [idx_s_v[pl.ds(i * 16, 16)]])
        o16 = plsc.load_gather(labels_v, [idx_o_v[pl.ds(i * 16, 16)]])
        pair_v[pl.ds(i * 16, 16)] = s16 * _NUM_OBJ_CLS + o16
    pltpu.async_copy(freq_hbm.at[pair_v], rows_v, sem).wait()
    pltpu.sync_copy(rel_hbm.at[pl.ds(base, _B_PER_W)], rel_v)

    def addbody(i, _):
        r = i // (_D_PAD // 16)
        c = (i % (_D_PAD // 16)) * 16
        rows_v[r, pl.ds(c, 16)] = (rows_v[r, pl.ds(c, 16)]
                                   + rel_v[r, pl.ds(c, 16)])
        return 0
    lax.fori_loop(0, _B_PER_W * (_D_PAD // 16), addbody, 0)
    pltpu.sync_copy(rows_v, out_hbm.at[pl.ds(base, _B_PER_W)])


def _sc_bias(subj, obj, labels, freq_pad, rel_pad):
    mesh = plsc.VectorSubcoreMesh(core_axis_name="c", subcore_axis_name="s")
    fn = functools.partial(
        pl.kernel,
        mesh=mesh,
        compiler_params=pltpu.CompilerParams(needs_layout_passes=False,
                                             use_tc_tiling_on_sc=False),
        out_type=jax.ShapeDtypeStruct((_R_PAD, _D_PAD), jnp.float32),
        scratch_types=[
            pltpu.VMEM((_B_PER_W,), jnp.int32),
            pltpu.VMEM((_B_PER_W,), jnp.int32),
            pltpu.VMEM((_B_PER_W,), jnp.int32),
            pltpu.VMEM((_L_PAD,), jnp.int32),
            pltpu.VMEM((_B_PER_W, _D_PAD), jnp.float32),
            pltpu.VMEM((_B_PER_W, _D_PAD), jnp.float32),
            pltpu.SemaphoreType.DMA,
        ],
    )(_sc_bias_body)
    return fn(subj, obj, labels, freq_pad, rel_pad)


def kernel(roi_features, union_features, rel_pair_idxs, boxes_per_cls,
           W_obj, W_rel, freq_bias):
    obj_logits = _matmul(roi_features, W_obj, block_rows=400)
    rel_logits = _matmul(union_features, W_rel, block_rows=600)
    pred_label = _nms(obj_logits, boxes_per_cls)

    subj = jnp.pad(rel_pair_idxs[:, 0], (0, _R_PAD - _N_REL))
    obj = jnp.pad(rel_pair_idxs[:, 1], (0, _R_PAD - _N_REL))
    freq_pad = jnp.pad(freq_bias, ((0, 0), (0, _D_PAD - _NUM_REL_CLS)))
    rel_pad = jnp.pad(rel_logits,
                      ((0, _R_PAD - _N_REL), (0, _D_PAD - _NUM_REL_CLS)))
    out = _sc_bias(subj, obj, pred_label, freq_pad, rel_pad)
    return out[:_N_REL, :_NUM_REL_CLS]


# speedup vs baseline: 47.4596x; 1.0006x over previous
import functools

import jax
import jax.numpy as jnp
from jax import lax
from jax.experimental import pallas as pl
from jax.experimental.pallas import tpu as pltpu
from jax.experimental.pallas import tpu_sc as plsc

_NUM_OBJ_CLS = 151
_NUM_REL_CLS = 51
_N_OBJ = 2000
_N_REL = 6000
_IN_CH = 4096
_NMS_THRESH = 0.5

_C_PAD = 152
_L_PAD = 2048
_R_PAD = 6144
_D_PAD = 64
_NEG = -1e30


def _mm_kernel(a_ref, w_ref, o_ref):
    o_ref[...] = jnp.dot(a_ref[...], w_ref[...],
                         preferred_element_type=jnp.float32)


def _mm_t_kernel(w_ref, a_ref, o_ref):
    o_ref[...] = lax.dot_general(w_ref[...], a_ref[...],
                                 (((0,), (1,)), ((), ())),
                                 preferred_element_type=jnp.float32)


def _matmul_t(w, a):
    k, m = w.shape
    n, _ = a.shape
    return pl.pallas_call(
        _mm_t_kernel,
        out_shape=jax.ShapeDtypeStruct((m, n), jnp.float32),
    )(w, a)


def _matmul(a, w, block_rows):
    n, k = a.shape
    _, m = w.shape
    grid = n // block_rows
    return pl.pallas_call(
        _mm_kernel,
        grid=(grid,),
        in_specs=[
            pl.BlockSpec((block_rows, k), lambda i: (i, 0)),
            pl.BlockSpec((k, m), lambda i: (0, 0)),
        ],
        out_specs=pl.BlockSpec((block_rows, m), lambda i: (i, 0)),
        out_shape=jax.ShapeDtypeStruct((n, m), jnp.float32),
    )(a, w)


_BIG = 3.0e7
_SL = 16
_LN = 128


def _nms_kernel(logits_t_ref, x1_ref, y1_ref, x2_ref, y2_ref,
                labels_ref, probs_ref, rm_ref, ra4_ref, stale_ref):
    cls_iota = lax.broadcasted_iota(jnp.int32, (_C_PAD, _SL, _LN), 0)
    box_iota = (lax.broadcasted_iota(jnp.int32, (_SL, _LN), 0) * _LN
                + lax.broadcasted_iota(jnp.int32, (_SL, _LN), 1))
    lane_key = box_iota.astype(jnp.float32) * 1024.0

    x = logits_t_ref[...]
    m0 = jnp.max(x, axis=0, keepdims=True)
    e = jnp.exp(x - m0)
    s = jnp.sum(e, axis=0, keepdims=True)
    p = e / s
    p = jnp.where(cls_iota == 0, 0.0, p)
    p = jnp.where(cls_iota >= _NUM_OBJ_CLS, 0.0, p)
    probs_ref[...] = p

    row_max0 = jnp.max(p, axis=0)
    rm0 = jnp.where(box_iota < _N_OBJ, row_max0, -1.0)
    rm_ref[...] = rm0
    ra4_0 = jnp.min(jnp.where(p == row_max0, cls_iota, _C_PAD),
                    axis=0).astype(jnp.float32) * 4.0
    ra4_ref[...] = ra4_0
    stale_ref[...] = jnp.zeros((_SL, _LN), jnp.float32)
    labels_ref[...] = jnp.zeros((_SL, _LN), jnp.int32)

    def pick(rm, key_extra):
        m = jnp.max(jnp.max(rm, axis=0, keepdims=True))
        kk = jnp.where(rm == m, key_extra, _BIG)
        return jnp.min(jnp.min(kk, axis=0, keepdims=True)).astype(jnp.int32)

    def fresh_pick(rm1):
        probs = probs_ref[...]
        rm = jnp.max(probs, axis=0)
        ra4 = jnp.min(jnp.where(probs == rm, cls_iota, _C_PAD),
                      axis=0).astype(jnp.float32) * 4.0
        rm = jnp.where(rm1 < 0.0, -1.0, rm)
        rm_ref[...] = rm
        ra4_ref[...] = ra4
        stale_ref[...] = jnp.zeros((_SL, _LN), jnp.float32)
        return pick(rm, lane_key + ra4)

    def body(_, bc):
        r = bc
        box = r >> 10
        rem = r - (box << 10)
        cls = rem >> 2
        is_box = box_iota == box

        col = probs_ref[pl.ds(cls, 1), :, :][0]
        rm1 = jnp.where(is_box, -1.0, rm_ref[...])
        rm_ref[...] = rm1
        candidate = (col >= rm1) & (rm1 > 0.0)
        stale_prev = stale_ref[...]
        flag = jnp.where(candidate, 1.0, stale_prev)
        nr = pick(rm1, lane_key + ra4_ref[...] + flag)

        x1 = x1_ref[pl.ds(cls, 1), :, :][0]
        y1 = y1_ref[pl.ds(cls, 1), :, :][0]
        x2 = x2_ref[pl.ds(cls, 1), :, :][0]
        y2 = y2_ref[pl.ds(cls, 1), :, :][0]
        stack = jnp.concatenate(
            [jnp.where(is_box, x1, _NEG)[None], jnp.where(is_box, y1, _NEG)[None],
             jnp.where(is_box, x2, _NEG)[None], jnp.where(is_box, y2, _NEG)[None]],
            axis=0)
        sel = jnp.max(jnp.max(stack, axis=1), axis=1, keepdims=True)
        sx1 = sel[0, 0]
        sy1 = sel[1, 0]
        sx2 = sel[2, 0]
        sy2 = sel[3, 0]

        ix = jnp.maximum(jnp.minimum(sx2, x2) - jnp.maximum(sx1, x1), 0.0)
        iy = jnp.maximum(jnp.minimum(sy2, y2) - jnp.maximum(sy1, y1), 0.0)
        inter = ix * iy
        a1 = jnp.maximum(sx2 - sx1, 0.0) * jnp.maximum(sy2 - sy1, 0.0)
        a2 = jnp.maximum(x2 - x1, 0.0) * jnp.maximum(y2 - y1, 0.0)
        iou = inter / (a1 + a2 - inter + 1e-9)
        overlap = iou > _NMS_THRESH

        probs_ref[pl.ds(cls, 1), :, :] = jnp.where(overlap, 0.0, col)[None]
        labels_ref[...] = jnp.where(is_box, cls, labels_ref[...])

        invalid = overlap & candidate
        stale_ref[...] = jnp.where(invalid, 1.0, stale_prev)

        return lax.cond((nr & 3) > 0, lambda _: fresh_pick(rm1),
                        lambda _: nr, 0)

    lax.fori_loop(0, _N_OBJ, body, pick(rm0, lane_key + ra4_0))


def _nms(obj_logits_t, boxes_per_cls):
    logits_t = jnp.pad(obj_logits_t,
                       ((0, _C_PAD - _NUM_OBJ_CLS), (0, _L_PAD - _N_OBJ)),
                       constant_values=_NEG).reshape(_C_PAD, _SL, _LN)
    bt = jnp.pad(boxes_per_cls.transpose(2, 1, 0),
                 ((0, 0), (0, _C_PAD - _NUM_OBJ_CLS), (0, _L_PAD - _N_OBJ))
                 ).reshape(4, _C_PAD, _SL, _LN)
    labels = pl.pallas_call(
        _nms_kernel,
        out_shape=jax.ShapeDtypeStruct((_SL, _LN), jnp.int32),
        scratch_shapes=[pltpu.VMEM((_C_PAD, _SL, _LN), jnp.float32),
                        pltpu.VMEM((_SL, _LN), jnp.float32),
                        pltpu.VMEM((_SL, _LN), jnp.float32),
                        pltpu.VMEM((_SL, _LN), jnp.float32)],
    )(logits_t, bt[0], bt[1], bt[2], bt[3])
    return labels.reshape(_L_PAD)


_B_PER_W = _R_PAD // 32


def _sc_bias_body(subj_hbm, obj_hbm, labels_hbm, freq_hbm, rel_hbm, out_hbm,
                  idx_s_v, idx_o_v, pair_v, labels_v, rows_v, rel_v, sem):
    wid = lax.axis_index("s") * 2 + lax.axis_index("c")
    base = wid * _B_PER_W
    pltpu.sync_copy(labels_hbm, labels_v)
    pltpu.sync_copy(subj_hbm.at[pl.ds(base, _B_PER_W)], idx_s_v)
    pltpu.sync_copy(obj_hbm.at[pl.ds(base, _B_PER_W)], idx_o_v)
    for i in range(_B_PER_W // 16):
        s16 = plsc.load_gather(labels_v, [idx_s_v[pl.ds(i * 16, 16)]])
        o16 = plsc.load_gather(labels_v, [idx_o_v[pl.ds(i * 16, 16)]])
        pair_v[pl.ds(i * 16, 16)] = s16 * _NUM_OBJ_CLS + o16
    pltpu.async_copy(freq_hbm.at[pair_v], rows_v, sem).wait()
    pltpu.sync_copy(rel_hbm.at[pl.ds(base, _B_PER_W)], rel_v)

    def addbody(i, _):
        r = i // (_D_PAD // 16)
        c = (i % (_D_PAD // 16)) * 16
        rows_v[r, pl.ds(c, 16)] = (rows_v[r, pl.ds(c, 16)]
                                   + rel_v[r, pl.ds(c, 16)])
        return 0
    lax.fori_loop(0, _B_PER_W * (_D_PAD // 16), addbody, 0)
    pltpu.sync_copy(rows_v, out_hbm.at[pl.ds(base, _B_PER_W)])


def _sc_bias(subj, obj, labels, freq_pad, rel_pad):
    mesh = plsc.VectorSubcoreMesh(core_axis_name="c", subcore_axis_name="s")
    fn = functools.partial(
        pl.kernel,
        mesh=mesh,
        compiler_params=pltpu.CompilerParams(needs_layout_passes=False,
                                             use_tc_tiling_on_sc=False),
        out_type=jax.ShapeDtypeStruct((_R_PAD, _D_PAD), jnp.float32),
        scratch_types=[
            pltpu.VMEM((_B_PER_W,), jnp.int32),
            pltpu.VMEM((_B_PER_W,), jnp.int32),
            pltpu.VMEM((_B_PER_W,), jnp.int32),
            pltpu.VMEM((_L_PAD,), jnp.int32),
            pltpu.VMEM((_B_PER_W, _D_PAD), jnp.float32),
            pltpu.VMEM((_B_PER_W, _D_PAD), jnp.float32),
            pltpu.SemaphoreType.DMA,
        ],
    )(_sc_bias_body)
    return fn(subj, obj, labels, freq_pad, rel_pad)


def kernel(roi_features, union_features, rel_pair_idxs, boxes_per_cls,
           W_obj, W_rel, freq_bias):
    obj_logits_t = _matmul_t(W_obj, roi_features)
    rel_logits = _matmul(union_features, W_rel, block_rows=600)
    pred_label = _nms(obj_logits_t, boxes_per_cls)

    subj = jnp.pad(rel_pair_idxs[:, 0], (0, _R_PAD - _N_REL))
    obj = jnp.pad(rel_pair_idxs[:, 1], (0, _R_PAD - _N_REL))
    freq_pad = jnp.pad(freq_bias, ((0, 0), (0, _D_PAD - _NUM_REL_CLS)))
    rel_pad = jnp.pad(rel_logits,
                      ((0, _R_PAD - _N_REL), (0, _D_PAD - _NUM_REL_CLS)))
    out = _sc_bias(subj, obj, pred_label, freq_pad, rel_pad)
    return out[:_N_REL, :_NUM_REL_CLS]
